# Initial kernel scaffold; baseline (speedup 1.0000x reference)
import jax
import jax.numpy as jnp
from jax.experimental import pallas as pl


def kernel(token_ids, embedding_layer):
    raise NotImplementedError("write your pallas kernel here")


# speedup vs baseline: 1.5732x; 1.5732x over previous
import jax
import jax.numpy as jnp
from jax import lax
from jax.experimental import pallas as pl
from jax.experimental.pallas import tpu as pltpu
from jax.experimental.pallas import tpu_sc as plsc

NUM_WORKERS = 32
WINDOW = 128


def kernel(token_ids, embedding_layer):
    n_rows, n_cols = token_ids.shape
    dim = embedding_layer.shape[1]
    num_indices = n_rows * n_cols
    idx = token_ids.reshape(num_indices)

    per_worker = num_indices // NUM_WORKERS
    n_chunks = per_worker // WINDOW

    mesh = plsc.VectorSubcoreMesh(core_axis_name="core",
                                  subcore_axis_name="subcore")

    @pl.kernel(
        out_type=jax.ShapeDtypeStruct((num_indices, dim), jnp.float32),
        mesh=mesh,
        compiler_params=pltpu.CompilerParams(use_tc_tiling_on_sc=False),
        scratch_types=[
            pltpu.VMEM((WINDOW,), jnp.int32),
            pltpu.VMEM((WINDOW, dim), jnp.float32),
            pltpu.SemaphoreType.DMA,
        ],
    )
    def gather_kernel(table_hbm, i_hbm, o_hbm, idx_v, rows_v, sem):
        wid = lax.axis_index("subcore") * 2 + lax.axis_index("core")
        base = wid * per_worker

        @pl.loop(0, n_chunks)
        def _(c):
            off = base + c * WINDOW
            pltpu.sync_copy(i_hbm.at[pl.ds(off, WINDOW)], idx_v)
            pltpu.async_copy(table_hbm.at[idx_v], rows_v, sem).wait()
            pltpu.sync_copy(rows_v, o_hbm.at[pl.ds(off, WINDOW)])

    out = gather_kernel(embedding_layer, idx)
    return out.reshape(n_rows, n_cols, dim)


# speedup vs baseline: 1.8740x; 1.1912x over previous
import jax
import jax.numpy as jnp
from jax import lax
from jax.experimental import pallas as pl
from jax.experimental.pallas import tpu as pltpu
from jax.experimental.pallas import tpu_sc as plsc

NUM_WORKERS = 32
WINDOW = 128
K = 8


def kernel(token_ids, embedding_layer):
    n_rows, n_cols = token_ids.shape
    dim = embedding_layer.shape[1]
    num_indices = n_rows * n_cols
    idx = token_ids.reshape(num_indices)

    per_worker = num_indices // NUM_WORKERS
    n_chunks = per_worker // WINDOW
    n_groups = n_chunks // K

    mesh = plsc.VectorSubcoreMesh(core_axis_name="core",
                                  subcore_axis_name="subcore")

    @pl.kernel(
        out_type=jax.ShapeDtypeStruct((num_indices, dim), jnp.float32),
        mesh=mesh,
        compiler_params=pltpu.CompilerParams(use_tc_tiling_on_sc=False),
        scratch_types=[
            pltpu.VMEM((per_worker,), jnp.int32),
            pltpu.VMEM((K, WINDOW, dim), jnp.float32),
            pltpu.SemaphoreType.DMA,
            pltpu.SemaphoreType.DMA,
        ],
    )
    def gather_kernel(table_hbm, i_hbm, o_hbm, idx_all, rows_v, gsem, wsem):
        wid = lax.axis_index("subcore") * 2 + lax.axis_index("core")
        base = wid * per_worker
        pltpu.sync_copy(i_hbm.at[pl.ds(base, per_worker)], idx_all)

        @pl.loop(0, n_groups)
        def _(grp):
            goff = grp * (K * WINDOW)
            gathers = []
            for j in range(K):
                loc = goff + j * WINDOW
                gathers.append(pltpu.async_copy(
                    table_hbm.at[idx_all.at[pl.ds(loc, WINDOW)]],
                    rows_v.at[j], gsem))
            writes = []
            for j in range(K):
                loc = goff + j * WINDOW
                gathers[j].wait()
                writes.append(pltpu.async_copy(
                    rows_v.at[j], o_hbm.at[pl.ds(base + loc, WINDOW)], wsem))
            for w in writes:
                w.wait()

    out = gather_kernel(embedding_layer, idx)
    return out.reshape(n_rows, n_cols, dim)
